# Initial kernel scaffold; baseline (speedup 1.0000x reference)
import jax
import jax.numpy as jnp
from jax.experimental import pallas as pl


def kernel(x, edge_index, W1, b1, W2, b2):
    raise NotImplementedError("write your pallas kernel here")


# speedup vs baseline: 8.3902x; 8.3902x over previous
import functools

import jax
import jax.numpy as jnp
from jax import lax
from jax.experimental import pallas as pl
from jax.experimental.pallas import tpu as pltpu
from jax.experimental.pallas import tpu_sc as plsc

_N = 10000
_E = 320000
_NPAD = 10112
_NC = 2
_NS = 16
_NW = _NC * _NS
_EW = _E // _NW
_B = 80
_NB = _EW // _B
_RW = _NPAD // _NS

@functools.lru_cache(maxsize=1)
def _get_mesh():
    return plsc.VectorSubcoreMesh(core_axis_name="c", subcore_axis_name="s")


def _deg_body(srcm_hbm, ones_hbm, zeros_hbm, out_hbm, idx_v, ones_v, acc, sem):
    c = lax.axis_index("c")
    s = lax.axis_index("s")
    wid = c * _NS + s
    r0 = pl.multiple_of(s * _RW, 8)
    pltpu.sync_copy(zeros_hbm.at[pl.ds(r0, _RW)], acc.at[pl.ds(r0, _RW)])
    pltpu.sync_copy(ones_hbm, ones_v)
    plsc.subcore_barrier()

    def body(i, carry):
        base = pl.multiple_of(wid * _EW + i * _B, 8)
        pltpu.sync_copy(srcm_hbm.at[pl.ds(base, _B)], idx_v)
        pltpu.sync_copy(ones_v, acc.at[idx_v], add=True)
        return carry

    lax.fori_loop(0, _NB, body, 0)
    plsc.subcore_barrier()
    pltpu.sync_copy(acc.at[pl.ds(r0, _RW)], out_hbm.at[c].at[pl.ds(r0, _RW)])


@jax.jit
def _sc_degree(srcm, ones, zeros):
    k = pl.kernel(
        _deg_body,
        mesh=_get_mesh(),
        compiler_params=pltpu.CompilerParams(use_tc_tiling_on_sc=False),
        out_type=jax.ShapeDtypeStruct((_NC, _NPAD, 16), jnp.float32),
        scratch_types=[
            pltpu.VMEM((_B,), jnp.int32),
            pltpu.VMEM((_B, 16), jnp.float32),
            pltpu.VMEM_SHARED((_NPAD, 16), jnp.float32),
            pltpu.SemaphoreType.DMA,
        ],
    )
    return k(srcm, ones, zeros)


def _scmul_body(t_hbm, gsrc_hbm, dst_hbm, zeros_hbm, out_hbm,
                sidx, didx, rows, acc, sem):
    c = lax.axis_index("c")
    s = lax.axis_index("s")
    wid = c * _NS + s
    r0 = pl.multiple_of(s * _RW, 8)
    pltpu.sync_copy(zeros_hbm.at[pl.ds(r0, _RW)], acc.at[pl.ds(r0, _RW)])
    plsc.subcore_barrier()

    def body(i, carry):
        base = pl.multiple_of(wid * _EW + i * _B, 8)
        pltpu.sync_copy(gsrc_hbm.at[pl.ds(base, _B)], sidx)
        pltpu.sync_copy(dst_hbm.at[pl.ds(base, _B)], didx)
        pltpu.async_copy(t_hbm.at[sidx], rows, sem).wait()
        pltpu.sync_copy(rows, acc.at[didx], add=True)
        return carry

    lax.fori_loop(0, _NB, body, 0)
    plsc.subcore_barrier()
    pltpu.sync_copy(acc.at[pl.ds(r0, _RW)], out_hbm.at[c].at[pl.ds(r0, _RW)])


@functools.partial(jax.jit, static_argnames=("f",))
def _sc_mul(t_pad, gsrc, dst, zeros, f):
    k = pl.kernel(
        _scmul_body,
        mesh=_get_mesh(),
        compiler_params=pltpu.CompilerParams(use_tc_tiling_on_sc=False),
        out_type=jax.ShapeDtypeStruct((_NC, _NPAD, f), jnp.float32),
        scratch_types=[
            pltpu.VMEM((_B,), jnp.int32),
            pltpu.VMEM((_B,), jnp.int32),
            pltpu.VMEM((_B, f), jnp.float32),
            pltpu.VMEM_SHARED((_NPAD, f), jnp.float32),
            pltpu.SemaphoreType.DMA,
        ],
    )
    return k(t_pad, gsrc, dst, zeros)


_TC_PARAMS = pltpu.CompilerParams(vmem_limit_bytes=110 * 1024 * 1024)


def _tc1_body(x_ref, w_ref, degp_ref, u0_ref, u2_ref, vcat_ref, dis_ref):
    x = x_ref[...]
    deg = degp_ref[0, :_N, 0:1] + degp_ref[1, :_N, 0:1]
    dis = jnp.where(deg > 0.0, lax.rsqrt(jnp.maximum(deg, 1e-30)), 0.0)
    u0 = jnp.dot(x, w_ref[0], preferred_element_type=jnp.float32)
    u1 = jnp.dot(x, w_ref[1], preferred_element_type=jnp.float32)
    u2 = jnp.dot(x, w_ref[2], preferred_element_type=jnp.float32)
    u0_ref[...] = u0
    u2_ref[...] = u2
    vcat_ref[...] = dis * jnp.concatenate([u1, u2], axis=1)
    dis_ref[...] = dis


@jax.jit
def _tc1(x, w1, deg_p):
    return pl.pallas_call(
        _tc1_body,
        compiler_params=_TC_PARAMS,
        out_shape=[
            jax.ShapeDtypeStruct((_N, 64), jnp.float32),
            jax.ShapeDtypeStruct((_N, 64), jnp.float32),
            jax.ShapeDtypeStruct((_N, 128), jnp.float32),
            jax.ShapeDtypeStruct((_N, 1), jnp.float32),
        ],
    )(x, w1, deg_p)


def _tc_mid_body(qp_ref, dis_ref, w2s_ref, *, f):
    dis = dis_ref[...]
    q2 = qp_ref[0, :_N, f:] + qp_ref[1, :_N, f:]
    w2s_ref[...] = (dis * dis) * q2


@functools.partial(jax.jit, static_argnames=("f",))
def _tc_mid(q_p, dis, f):
    return pl.pallas_call(
        functools.partial(_tc_mid_body, f=f),
        compiler_params=_TC_PARAMS,
        out_shape=jax.ShapeDtypeStruct((_N, f), jnp.float32),
    )(q_p, dis)


def _tc3_body(u0_ref, u2_ref, qp_ref, rp_ref, dis_ref, b1_ref, w2_ref,
              u0p_ref, u2p_ref, vcatp_ref):
    dis = dis_ref[...]
    q1 = qp_ref[0, :_N, :64] + qp_ref[1, :_N, :64]
    r2 = rp_ref[0, :_N, :] + rp_ref[1, :_N, :]
    out1 = (u0_ref[...] - dis * q1 + 2.0 * (dis * r2) - u2_ref[...]
            + b1_ref[...])
    h = jnp.maximum(out1, 0.0)
    u0p = jnp.dot(h, w2_ref[0], preferred_element_type=jnp.float32)
    u1p = jnp.dot(h, w2_ref[1], preferred_element_type=jnp.float32)
    u2p = jnp.dot(h, w2_ref[2], preferred_element_type=jnp.float32)
    u0p_ref[...] = u0p
    u2p_ref[...] = u2p
    vcatp_ref[...] = dis * jnp.concatenate([u1p, u2p], axis=1)


@jax.jit
def _tc3(u0, u2, q_p, r_p, dis, b1, w2):
    return pl.pallas_call(
        _tc3_body,
        compiler_params=_TC_PARAMS,
        out_shape=[
            jax.ShapeDtypeStruct((_N, 16), jnp.float32),
            jax.ShapeDtypeStruct((_N, 16), jnp.float32),
            jax.ShapeDtypeStruct((_N, 32), jnp.float32),
        ],
    )(u0, u2, q_p, r_p, dis, b1[None, :], w2)


def _tc5_body(u0p_ref, u2p_ref, qp_ref, rp_ref, dis_ref, b2_ref, out_ref):
    dis = dis_ref[...]
    q1 = qp_ref[0, :_N, :16] + qp_ref[1, :_N, :16]
    r2 = rp_ref[0, :_N, :] + rp_ref[1, :_N, :]
    out_ref[...] = (u0p_ref[...] - dis * q1 + 2.0 * (dis * r2)
                    - u2p_ref[...] + b2_ref[...])


@jax.jit
def _tc5(u0p, u2p, qp_p, rp_p, dis, b2):
    return pl.pallas_call(
        _tc5_body,
        compiler_params=_TC_PARAMS,
        out_shape=jax.ShapeDtypeStruct((_N, 16), jnp.float32),
    )(u0p, u2p, qp_p, rp_p, dis, b2[None, :])


def _pad_rows(m):
    return jnp.pad(m, ((0, _NPAD - _N), (0, 0)))


def kernel(x, edge_index, W1, b1, W2, b2):
    src = edge_index[0]
    dst = edge_index[1]
    gsrc = jnp.where(src != dst, src, _N)

    ones16 = jnp.ones((_B, 16), jnp.float32)
    z16 = jnp.zeros((_NPAD, 16), jnp.float32)
    deg_p = _sc_degree(gsrc, ones16, z16)

    u0, u2, vcat, dis = _tc1(x, W1, deg_p)

    z128 = jnp.zeros((_NPAD, 128), jnp.float32)
    z64 = z128[:, :64]
    z32 = z128[:, :32]
    q_p = _sc_mul(_pad_rows(vcat), gsrc, dst, z128, 128)
    w2s = _tc_mid(q_p, dis, 64)
    r_p = _sc_mul(_pad_rows(w2s), gsrc, dst, z64, 64)

    u0p, u2p, vcatp = _tc3(u0, u2, q_p, r_p, dis, b1, W2)

    qp_p = _sc_mul(_pad_rows(vcatp), gsrc, dst, z32, 32)
    w2sp = _tc_mid(qp_p, dis, 16)
    z16f = z128[:, :16]
    rp_p = _sc_mul(_pad_rows(w2sp), gsrc, dst, z16f, 16)

    return _tc5(u0p, u2p, qp_p, rp_p, dis, b2)


# speedup vs baseline: 20.0858x; 2.3940x over previous
import functools

import jax
import jax.numpy as jnp
from jax import lax
from jax.experimental import pallas as pl
from jax.experimental.pallas import tpu as pltpu
from jax.experimental.pallas import tpu_sc as plsc

_N = 10000
_E = 320000
_NPAD = 10112
_NC = 2
_NS = 16
_NW = _NC * _NS
_EW = _E // _NW
_B = 80
_NB = _EW // _B
_RW = _NPAD // _NS

@functools.lru_cache(maxsize=1)
def _get_mesh():
    return plsc.VectorSubcoreMesh(core_axis_name="c", subcore_axis_name="s")


def _deg_body(srcm_hbm, ones_hbm, zeros_hbm, out_hbm, idx_v, ones_v, acc, sem):
    c = lax.axis_index("c")
    s = lax.axis_index("s")
    wid = c * _NS + s
    r0 = pl.multiple_of(s * _RW, 8)
    pltpu.sync_copy(zeros_hbm.at[pl.ds(r0, _RW)], acc.at[pl.ds(r0, _RW)])
    pltpu.sync_copy(srcm_hbm.at[wid], idx_v)
    pltpu.sync_copy(ones_hbm, ones_v)
    plsc.subcore_barrier()

    def body(j, carry):
        pltpu.sync_copy(ones_v, acc.at[idx_v.at[j]], add=True)
        return carry

    lax.fori_loop(0, _NB, body, 0)
    plsc.subcore_barrier()
    pltpu.sync_copy(acc.at[pl.ds(r0, _RW)], out_hbm.at[c].at[pl.ds(r0, _RW)])


@jax.jit
def _sc_degree(srcm, ones, zeros):
    k = pl.kernel(
        _deg_body,
        mesh=_get_mesh(),
        compiler_params=pltpu.CompilerParams(use_tc_tiling_on_sc=False),
        out_type=jax.ShapeDtypeStruct((_NC, _NPAD, 16), jnp.float32),
        scratch_types=[
            pltpu.VMEM((_NB, _B), jnp.int32),
            pltpu.VMEM((_B, 16), jnp.float32),
            pltpu.VMEM_SHARED((_NPAD, 16), jnp.float32),
            pltpu.SemaphoreType.DMA,
        ],
    )
    return k(srcm, ones, zeros)


def _scmul_body(t_hbm, gsrc_hbm, dst_hbm, zeros_hbm, out_hbm,
                sidx, didx, rows0, rows1, acc, sem0, sem1):
    c = lax.axis_index("c")
    s = lax.axis_index("s")
    wid = c * _NS + s
    r0 = pl.multiple_of(s * _RW, 8)
    pltpu.sync_copy(zeros_hbm.at[pl.ds(r0, _RW)], acc.at[pl.ds(r0, _RW)])
    pltpu.sync_copy(gsrc_hbm.at[wid], sidx)
    pltpu.sync_copy(dst_hbm.at[wid], didx)
    plsc.subcore_barrier()

    pltpu.async_copy(t_hbm.at[sidx.at[0]], rows0, sem0)

    def wait0():
        pltpu.make_async_copy(t_hbm.at[sidx.at[0]], rows0, sem0).wait()

    def wait1():
        pltpu.make_async_copy(t_hbm.at[sidx.at[0]], rows1, sem1).wait()

    def body(k, carry):
        j = 2 * k
        pltpu.async_copy(t_hbm.at[sidx.at[j + 1]], rows1, sem1)
        wait0()
        pltpu.sync_copy(rows0, acc.at[didx.at[j]], add=True)
        pltpu.async_copy(t_hbm.at[sidx.at[j + 2]], rows0, sem0)
        wait1()
        pltpu.sync_copy(rows1, acc.at[didx.at[j + 1]], add=True)
        return carry

    lax.fori_loop(0, (_NB - 1) // 2, body, 0)
    wait0()
    pltpu.sync_copy(rows0, acc.at[didx.at[_NB - 1]], add=True)

    plsc.subcore_barrier()
    pltpu.sync_copy(acc.at[pl.ds(r0, _RW)], out_hbm.at[c].at[pl.ds(r0, _RW)])


@functools.partial(jax.jit, static_argnames=("f",))
def _sc_mul(t_pad, gsrc, dst, zeros, f):
    k = pl.kernel(
        _scmul_body,
        mesh=_get_mesh(),
        compiler_params=pltpu.CompilerParams(use_tc_tiling_on_sc=False),
        out_type=jax.ShapeDtypeStruct((_NC, _NPAD, f), jnp.float32),
        scratch_types=[
            pltpu.VMEM((_NB, _B), jnp.int32),
            pltpu.VMEM((_NB, _B), jnp.int32),
            pltpu.VMEM((_B, f), jnp.float32),
            pltpu.VMEM((_B, f), jnp.float32),
            pltpu.VMEM_SHARED((_NPAD, f), jnp.float32),
            pltpu.SemaphoreType.DMA,
            pltpu.SemaphoreType.DMA,
        ],
    )
    return k(t_pad, gsrc, dst, zeros)


_TC_PARAMS = pltpu.CompilerParams(vmem_limit_bytes=110 * 1024 * 1024)


def _tc1_body(x_ref, w_ref, degp_ref, u0_ref, u2_ref, vcat_ref, dis_ref):
    x = x_ref[...]
    deg = degp_ref[0, :_N, 0:1] + degp_ref[1, :_N, 0:1]
    dis = jnp.where(deg > 0.0, lax.rsqrt(jnp.maximum(deg, 1e-30)), 0.0)
    u0 = jnp.dot(x, w_ref[0], preferred_element_type=jnp.float32)
    u1 = jnp.dot(x, w_ref[1], preferred_element_type=jnp.float32)
    u2 = jnp.dot(x, w_ref[2], preferred_element_type=jnp.float32)
    u0_ref[...] = u0
    u2_ref[...] = u2
    vcat = dis * jnp.concatenate([u1, u2], axis=1)
    vcat_ref[...] = jnp.pad(vcat, ((0, _NPAD - _N), (0, 0)))
    dis_ref[...] = dis


@jax.jit
def _tc1(x, w1, deg_p):
    return pl.pallas_call(
        _tc1_body,
        compiler_params=_TC_PARAMS,
        out_shape=[
            jax.ShapeDtypeStruct((_N, 64), jnp.float32),
            jax.ShapeDtypeStruct((_N, 64), jnp.float32),
            jax.ShapeDtypeStruct((_NPAD, 128), jnp.float32),
            jax.ShapeDtypeStruct((_N, 1), jnp.float32),
        ],
    )(x, w1, deg_p)


def _tc_mid_body(qp_ref, dis_ref, w2s_ref, *, f):
    dis = dis_ref[...]
    q2 = qp_ref[0, :_N, f:] + qp_ref[1, :_N, f:]
    w2s_ref[...] = jnp.pad((dis * dis) * q2, ((0, _NPAD - _N), (0, 0)))


@functools.partial(jax.jit, static_argnames=("f",))
def _tc_mid(q_p, dis, f):
    return pl.pallas_call(
        functools.partial(_tc_mid_body, f=f),
        compiler_params=_TC_PARAMS,
        out_shape=jax.ShapeDtypeStruct((_NPAD, f), jnp.float32),
    )(q_p, dis)


def _tc3_body(u0_ref, u2_ref, qp_ref, rp_ref, dis_ref, b1_ref, w2_ref,
              u0p_ref, u2p_ref, vcatp_ref):
    dis = dis_ref[...]
    q1 = qp_ref[0, :_N, :64] + qp_ref[1, :_N, :64]
    r2 = rp_ref[0, :_N, :] + rp_ref[1, :_N, :]
    out1 = (u0_ref[...] - dis * q1 + 2.0 * (dis * r2) - u2_ref[...]
            + b1_ref[...])
    h = jnp.maximum(out1, 0.0)
    u0p = jnp.dot(h, w2_ref[0], preferred_element_type=jnp.float32)
    u1p = jnp.dot(h, w2_ref[1], preferred_element_type=jnp.float32)
    u2p = jnp.dot(h, w2_ref[2], preferred_element_type=jnp.float32)
    u0p_ref[...] = u0p
    u2p_ref[...] = u2p
    vcatp = dis * jnp.concatenate([u1p, u2p], axis=1)
    vcatp_ref[...] = jnp.pad(vcatp, ((0, _NPAD - _N), (0, 0)))


@jax.jit
def _tc3(u0, u2, q_p, r_p, dis, b1, w2):
    return pl.pallas_call(
        _tc3_body,
        compiler_params=_TC_PARAMS,
        out_shape=[
            jax.ShapeDtypeStruct((_N, 16), jnp.float32),
            jax.ShapeDtypeStruct((_N, 16), jnp.float32),
            jax.ShapeDtypeStruct((_NPAD, 32), jnp.float32),
        ],
    )(u0, u2, q_p, r_p, dis, b1[None, :], w2)


def _tc5_body(u0p_ref, u2p_ref, qp_ref, rp_ref, dis_ref, b2_ref, out_ref):
    dis = dis_ref[...]
    q1 = qp_ref[0, :_N, :16] + qp_ref[1, :_N, :16]
    r2 = rp_ref[0, :_N, :] + rp_ref[1, :_N, :]
    out_ref[...] = (u0p_ref[...] - dis * q1 + 2.0 * (dis * r2)
                    - u2p_ref[...] + b2_ref[...])


@jax.jit
def _tc5(u0p, u2p, qp_p, rp_p, dis, b2):
    return pl.pallas_call(
        _tc5_body,
        compiler_params=_TC_PARAMS,
        out_shape=jax.ShapeDtypeStruct((_N, 16), jnp.float32),
    )(u0p, u2p, qp_p, rp_p, dis, b2[None, :])


def kernel(x, edge_index, W1, b1, W2, b2):
    src = edge_index[0]
    dst = edge_index[1]
    gsrc = jnp.where(src != dst, src, _N)
    gsrc3 = gsrc.reshape(_NW, _NB, _B)
    dst3 = dst.reshape(_NW, _NB, _B)

    ones16 = jnp.ones((_B, 16), jnp.float32)
    z16 = jnp.zeros((_NPAD, 16), jnp.float32)
    deg_p = _sc_degree(gsrc3, ones16, z16)

    u0, u2, vcat, dis = _tc1(x, W1, deg_p)

    z128 = jnp.zeros((_NPAD, 128), jnp.float32)
    z64 = z128[:, :64]
    z32 = z128[:, :32]
    z16f = z128[:, :16]
    q_p = _sc_mul(vcat, gsrc3, dst3, z128, 128)
    w2s = _tc_mid(q_p, dis, 64)
    r_p = _sc_mul(w2s, gsrc3, dst3, z64, 64)

    u0p, u2p, vcatp = _tc3(u0, u2, q_p, r_p, dis, b1, W2)

    qp_p = _sc_mul(vcatp, gsrc3, dst3, z32, 32)
    w2sp = _tc_mid(qp_p, dis, 16)
    rp_p = _sc_mul(w2sp, gsrc3, dst3, z16f, 16)

    return _tc5(u0p, u2p, qp_p, rp_p, dis, b2)
